# Initial kernel scaffold; baseline (speedup 1.0000x reference)
import jax
import jax.numpy as jnp
from jax.experimental import pallas as pl


def kernel(edge_feat, emb_table, lin_W, lin_b):
    raise NotImplementedError("write your pallas kernel here")


# speedup vs baseline: 4.0930x; 4.0930x over previous
import functools

import jax
import jax.numpy as jnp
from jax import lax
from jax.experimental import pallas as pl
from jax.experimental.pallas import tpu as pltpu
from jax.experimental.pallas import tpu_sc as plsc

_L = 16


def _build_sc_call(E, V, H, n_workers):
    assert E % n_workers == 0
    ew = E // n_workers
    C = 80
    assert ew % C == 0
    n_chunks = ew // C
    assert H % _L == 0
    hc = H // _L

    mesh = plsc.VectorSubcoreMesh(core_axis_name="c", subcore_axis_name="s")

    @functools.partial(
        pl.kernel,
        mesh=mesh,
        out_type=jax.ShapeDtypeStruct((E, H), jnp.float32),
        scratch_types=[
            pltpu.VMEM((C,), jnp.int32),
            pltpu.VMEM((C,), jnp.int32),
            pltpu.VMEM((C,), jnp.int32),
            pltpu.VMEM((C,), jnp.int32),
            pltpu.VMEM((C,), jnp.float32),
            pltpu.VMEM((C, H), jnp.float32),
            pltpu.VMEM((C, H), jnp.float32),
            pltpu.VMEM((H,), jnp.float32),
            pltpu.VMEM((H,), jnp.float32),
            pltpu.SemaphoreType.DMA,
            pltpu.SemaphoreType.DMA,
        ],
    )
    def sc_call(idx0_hbm, idx1_hbm, c2_hbm, c3_hbm, table_hbm, w_hbm, b_hbm,
                out_hbm,
                idx0_v, idx1_v, c2_v, c3_v, s_v, rows0_v, rows1_v,
                w_v, b2_v, sem0, sem1):
        nc = 2
        wid = lax.axis_index("s") * nc + lax.axis_index("c")
        base = wid * ew

        pltpu.sync_copy(w_hbm, w_v)
        pltpu.sync_copy(b_hbm, b2_v)
        for h in range(hc):
            hs = pl.ds(h * _L, _L)
            b2_v[hs] = b2_v[hs] * 2.0

        def chunk_body(g, carry):
            cbase = base + g * C
            pltpu.sync_copy(idx0_hbm.at[pl.ds(cbase, C)], idx0_v)
            pltpu.sync_copy(idx1_hbm.at[pl.ds(cbase, C)], idx1_v)
            pltpu.sync_copy(c2_hbm.at[pl.ds(cbase, C)], c2_v)
            pltpu.sync_copy(c3_hbm.at[pl.ds(cbase, C)], c3_v)
            cp0 = pltpu.async_copy(table_hbm.at[idx0_v], rows0_v, sem0)
            cp1 = pltpu.async_copy(table_hbm.at[idx1_v], rows1_v, sem1)
            for j in range(C // _L):
                js = pl.ds(j * _L, _L)
                s_v[js] = (c2_v[js] + c3_v[js]).astype(jnp.float32)
            cp0.wait()
            cp1.wait()

            def group_body(g2, carry2):
                e0 = g2 * _L
                sv = s_v[pl.ds(e0, _L)]
                for k in range(_L):
                    sb = sv[k]
                    for h in range(hc):
                        hs = pl.ds(h * _L, _L)
                        rows0_v[e0 + k, hs] = (rows0_v[e0 + k, hs]
                                               + rows1_v[e0 + k, hs]
                                               + sb * w_v[hs] + b2_v[hs])
                return carry2

            lax.fori_loop(0, C // _L, group_body, 0, unroll=False)
            pltpu.sync_copy(rows0_v, out_hbm.at[pl.ds(cbase, C)])
            return carry

        lax.fori_loop(0, n_chunks, chunk_body, 0, unroll=False)

    return sc_call


@jax.jit
def _run(idx0, idx1, c2, c3, emb_table, w, b):
    E = idx0.shape[0]
    V, H = emb_table.shape
    sc_call = _build_sc_call(E, V, H, 32)
    return sc_call(idx0, idx1, c2, c3, emb_table, w, b)


def kernel(edge_feat, emb_table, lin_W, lin_b):
    E, F = edge_feat.shape
    ef = edge_feat.astype(jnp.int32)
    w = lin_W[:, 0]
    out = _run(ef[:, 0], ef[:, 1], ef[:, 2], ef[:, 3], emb_table, w, lin_b)
    return out.reshape(1, E, emb_table.shape[1])


# speedup vs baseline: 6.8374x; 1.6705x over previous
import functools

import jax
import jax.numpy as jnp
from jax import lax
from jax.experimental import pallas as pl
from jax.experimental.pallas import tpu as pltpu
from jax.experimental.pallas import tpu_sc as plsc

_L = 16


def _build_sc_call(E, V, H, n_workers):
    assert E % n_workers == 0
    ew = E // n_workers
    C = 80
    assert ew % C == 0
    n_chunks = ew // C
    assert H % _L == 0
    hc = H // _L

    mesh = plsc.VectorSubcoreMesh(core_axis_name="c", subcore_axis_name="s")

    @functools.partial(
        pl.kernel,
        mesh=mesh,
        out_type=jax.ShapeDtypeStruct((E, H), jnp.float32),
        scratch_types=[
            pltpu.VMEM((ew,), jnp.int32),
            pltpu.VMEM((ew,), jnp.int32),
            pltpu.VMEM((ew,), jnp.int32),
            pltpu.VMEM((ew,), jnp.int32),
            pltpu.VMEM((ew,), jnp.float32),
            pltpu.VMEM((C, H), jnp.float32),
            pltpu.VMEM((C, H), jnp.float32),
            pltpu.VMEM((C, H), jnp.float32),
            pltpu.VMEM((C, H), jnp.float32),
            pltpu.VMEM((C, H), jnp.float32),
            pltpu.VMEM((C, H), jnp.float32),
            pltpu.VMEM((H,), jnp.float32),
            pltpu.VMEM((H,), jnp.float32),
            pltpu.SemaphoreType.DMA,
            pltpu.SemaphoreType.DMA,
            pltpu.SemaphoreType.DMA,
            pltpu.SemaphoreType.DMA,
            pltpu.SemaphoreType.DMA,
            pltpu.SemaphoreType.DMA,
        ],
    )
    def sc_call(idx0_hbm, idx1_hbm, c2_hbm, c3_hbm, table_hbm, w_hbm, b_hbm,
                out_hbm,
                idx0_v, idx1_v, c2_v, c3_v, s_v,
                rows0a, rows0b, rows1a, rows1b, outa, outb,
                w_v, b2_v,
                gs0a, gs0b, gs1a, gs1b, osa, osb):
        rows0 = (rows0a, rows0b)
        rows1 = (rows1a, rows1b)
        outst = (outa, outb)
        gs0 = (gs0a, gs0b)
        gs1 = (gs1a, gs1b)
        osem = (osa, osb)

        nc = 2
        wid = lax.axis_index("s") * nc + lax.axis_index("c")
        base = wid * ew

        pltpu.sync_copy(w_hbm, w_v)
        pltpu.sync_copy(b_hbm, b2_v)
        pltpu.sync_copy(idx0_hbm.at[pl.ds(base, ew)], idx0_v)
        pltpu.sync_copy(idx1_hbm.at[pl.ds(base, ew)], idx1_v)
        pltpu.sync_copy(c2_hbm.at[pl.ds(base, ew)], c2_v)
        pltpu.sync_copy(c3_hbm.at[pl.ds(base, ew)], c3_v)
        for h in range(hc):
            hs = pl.ds(h * _L, _L)
            b2_v[hs] = b2_v[hs] * 2.0

        @plsc.parallel_loop(0, ew, _L)
        def _(j):
            js = pl.ds(j, _L)
            s_v[js] = (c2_v[js] + c3_v[js]).astype(jnp.float32)

        def gathers(g, p):
            loc = g * C
            cp0 = pltpu.make_async_copy(
                table_hbm.at[idx0_v.at[pl.ds(loc, C)]], rows0[p], gs0[p])
            cp1 = pltpu.make_async_copy(
                table_hbm.at[idx1_v.at[pl.ds(loc, C)]], rows1[p], gs1[p])
            return cp0, cp1

        def out_copy(g, p):
            return pltpu.make_async_copy(
                outst[p], out_hbm.at[pl.ds(base + g * C, C)], osem[p])

        def compute(g, p):
            loc = g * C
            r0 = rows0[p]
            r1 = rows1[p]
            ot = outst[p]

            @plsc.parallel_loop(0, C, _L)
            def _(e0):
                sv = s_v[pl.ds(loc + e0, _L)]
                for k in range(_L):
                    sb = sv[k]
                    for h in range(hc):
                        hs = pl.ds(h * _L, _L)
                        ot[e0 + k, hs] = (r0[e0 + k, hs] + r1[e0 + k, hs]
                                          + sb * w_v[hs] + b2_v[hs])

        def iteration(g, p):
            cn0, cn1 = gathers(g + 1, 1 - p)
            cn0.start()
            cn1.start()
            c0, c1 = gathers(g, p)
            c0.wait()
            c1.wait()

            @pl.when(g >= 2)
            def _():
                out_copy(g - 2, p).wait()

            compute(g, p)
            out_copy(g, p).start()

        i0, i1 = gathers(0, 0)
        i0.start()
        i1.start()

        def pair_body(t, carry):
            iteration(2 * t, 0)
            iteration(2 * t + 1, 1)
            return carry

        lax.fori_loop(0, (n_chunks - 1) // 2, pair_body, 0, unroll=False)

        gl = n_chunks - 1
        c0, c1 = gathers(gl, 0)
        c0.wait()
        c1.wait()
        out_copy(gl - 2, 0).wait()
        compute(gl, 0)
        out_copy(gl, 0).start()
        out_copy(gl - 1, 1).wait()
        out_copy(gl, 0).wait()

    return sc_call


@jax.jit
def _run(idx0, idx1, c2, c3, emb_table, w, b):
    E = idx0.shape[0]
    V, H = emb_table.shape
    sc_call = _build_sc_call(E, V, H, 32)
    return sc_call(idx0, idx1, c2, c3, emb_table, w, b)


def kernel(edge_feat, emb_table, lin_W, lin_b):
    E, F = edge_feat.shape
    ef = edge_feat.astype(jnp.int32)
    w = lin_W[:, 0]
    out = _run(ef[:, 0], ef[:, 1], ef[:, 2], ef[:, 3], emb_table, w, lin_b)
    return out.reshape(1, E, emb_table.shape[1])
